# Initial kernel scaffold; baseline (speedup 1.0000x reference)
import functools

import jax
import jax.numpy as jnp
from jax import lax
from jax.experimental import pallas as pl
from jax.experimental.pallas import tpu as pltpu
from jax.experimental.pallas import tpu_sc as plsc

NC = 2
NS = 16
CHUNK = 128


def _mesh():
    return plsc.VectorSubcoreMesh(core_axis_name="c", subcore_axis_name="s")


@functools.lru_cache(maxsize=None)
def _edge_attr_agg(N, E, F):
    nchunks = E // CHUNK
    NW = NC * NS
    npt = N // NS

    @functools.partial(
        pl.kernel,
        mesh=_mesh(),
        out_type=(
            jax.ShapeDtypeStruct((NC * N, F), jnp.float32),
            jax.ShapeDtypeStruct((NC * N,), jnp.float32),
        ),
        scratch_types=[
            pltpu.VMEM((CHUNK,), jnp.int32),
            pltpu.VMEM((CHUNK, F), jnp.float32),
            pltpu.VMEM((CHUNK,), jnp.float32),
            pltpu.VMEM_SHARED((N, F), jnp.float32),
            pltpu.VMEM_SHARED((N,), jnp.float32),
        ],
    )
    def k(tgt_hbm, attr_hbm, zF_hbm, z1_hbm, outF_hbm, outC_hbm,
          idx_v, rows_v, ones_v, accF, accC):
        c = lax.axis_index("c")
        s = lax.axis_index("s")
        w = s * NC + c
        r0 = s * npt
        pltpu.sync_copy(zF_hbm.at[pl.ds(r0, npt)], accF.at[pl.ds(r0, npt)])
        pltpu.sync_copy(z1_hbm.at[pl.ds(r0, npt)], accC.at[pl.ds(r0, npt)])
        for i in range(CHUNK // 16):
            ones_v[pl.ds(i * 16, 16)] = jnp.full((16,), 1.0, jnp.float32)
        plsc.subcore_barrier()

        nmine = (nchunks - w + NW - 1) // NW

        def body(jj, carry):
            base = (w + jj * NW) * CHUNK
            pltpu.sync_copy(tgt_hbm.at[pl.ds(base, CHUNK)], idx_v)
            pltpu.sync_copy(attr_hbm.at[pl.ds(base, CHUNK)], rows_v)
            pltpu.sync_copy(rows_v, accF.at[idx_v], add=True)
            pltpu.sync_copy(ones_v, accC.at[idx_v], add=True)
            return carry

        lax.fori_loop(0, nmine, body, 0)
        plsc.subcore_barrier()
        pltpu.sync_copy(accF.at[pl.ds(r0, npt)],
                        outF_hbm.at[pl.ds(c * N + r0, npt)])
        pltpu.sync_copy(accC.at[pl.ds(r0, npt)],
                        outC_hbm.at[pl.ds(c * N + r0, npt)])

    return k


@functools.lru_cache(maxsize=None)
def _x_gather_scatter(N, E, H):
    nchunks = E // CHUNK
    npt = N // NS

    @functools.partial(
        pl.kernel,
        mesh=_mesh(),
        out_type=jax.ShapeDtypeStruct((NC * N, H), jnp.float32),
        scratch_types=[
            pltpu.VMEM((CHUNK,), jnp.int32),
            pltpu.VMEM((CHUNK,), jnp.int32),
            pltpu.VMEM((CHUNK, H), jnp.float32),
            pltpu.SemaphoreType.DMA,
            pltpu.VMEM_SHARED((N, H), jnp.float32),
        ],
    )
    def k(xs_hbm, src_hbm, tgt_hbm, z_hbm, out_hbm,
          src_v, tgt_v, rows_v, sem, acc):
        c = lax.axis_index("c")
        s = lax.axis_index("s")
        r0 = s * npt
        pltpu.sync_copy(z_hbm.at[pl.ds(r0, npt)], acc.at[pl.ds(r0, npt)])
        plsc.subcore_barrier()

        off = c * N
        nmine = (nchunks - s + NS - 1) // NS

        def body(jj, carry):
            base = (s + jj * NS) * CHUNK
            pltpu.sync_copy(src_hbm.at[pl.ds(base, CHUNK)], src_v)
            for i in range(CHUNK // 16):
                sl = pl.ds(i * 16, 16)
                src_v[sl] = src_v[sl] + off
            pltpu.async_copy(xs_hbm.at[src_v], rows_v, sem).wait()
            pltpu.sync_copy(tgt_hbm.at[pl.ds(base, CHUNK)], tgt_v)
            pltpu.sync_copy(rows_v, acc.at[tgt_v], add=True)
            return carry

        lax.fori_loop(0, nmine, body, 0)
        plsc.subcore_barrier()
        pltpu.sync_copy(acc.at[pl.ds(r0, npt)],
                        out_hbm.at[pl.ds(c * N + r0, npt)])

    return k


@functools.lru_cache(maxsize=None)
def _ae_gather_scatter(N, E, F):
    nchunks = E // CHUNK
    NW = NC * NS
    npt = N // NS

    @functools.partial(
        pl.kernel,
        mesh=_mesh(),
        out_type=jax.ShapeDtypeStruct((NC * N, F), jnp.float32),
        scratch_types=[
            pltpu.VMEM((CHUNK,), jnp.int32),
            pltpu.VMEM((CHUNK,), jnp.int32),
            pltpu.VMEM((CHUNK, F), jnp.float32),
            pltpu.SemaphoreType.DMA,
            pltpu.VMEM_SHARED((N, F), jnp.float32),
        ],
    )
    def k(ae_hbm, src_hbm, tgt_hbm, z_hbm, out_hbm,
          src_v, tgt_v, rows_v, sem, acc):
        c = lax.axis_index("c")
        s = lax.axis_index("s")
        w = s * NC + c
        r0 = s * npt
        pltpu.sync_copy(z_hbm.at[pl.ds(r0, npt)], acc.at[pl.ds(r0, npt)])
        plsc.subcore_barrier()

        nmine = (nchunks - w + NW - 1) // NW

        def body(jj, carry):
            base = (w + jj * NW) * CHUNK
            pltpu.sync_copy(src_hbm.at[pl.ds(base, CHUNK)], src_v)
            pltpu.async_copy(ae_hbm.at[src_v], rows_v, sem).wait()
            pltpu.sync_copy(tgt_hbm.at[pl.ds(base, CHUNK)], tgt_v)
            pltpu.sync_copy(rows_v, acc.at[tgt_v], add=True)
            return carry

        lax.fori_loop(0, nmine, body, 0)
        plsc.subcore_barrier()
        pltpu.sync_copy(acc.at[pl.ds(r0, npt)],
                        out_hbm.at[pl.ds(c * N + r0, npt)])

    return k


def _mean_body(aes_ref, cnt_ref, ae_ref, rc_ref):
    c = cnt_ref[0] + cnt_ref[1]
    rc = 1.0 / jnp.maximum(c, 1.0)
    rc_ref[...] = rc
    ae_ref[...] = (aes_ref[0] + aes_ref[1]) * rc


@functools.lru_cache(maxsize=None)
def _mean_kernel(N, F, BN):
    nb = N // BN
    return pl.pallas_call(
        _mean_body,
        grid=(nb,),
        in_specs=[
            pl.BlockSpec((2, BN, F), lambda i: (0, i, 0)),
            pl.BlockSpec((2, BN, 1), lambda i: (0, i, 0)),
        ],
        out_specs=[
            pl.BlockSpec((BN, F), lambda i: (i, 0)),
            pl.BlockSpec((BN, 1), lambda i: (i, 0)),
        ],
        out_shape=[
            jax.ShapeDtypeStruct((N, F), jnp.float32),
            jax.ShapeDtypeStruct((N, 1), jnp.float32),
        ],
    )


def _dense_body(a1_ref, a2_ref, ae_ref, rc_ref, x_ref,
                wlx_ref, wle_ref, wrx_ref, wre_ref,
                bl_ref, g_ref, b_ref, out_ref):
    a1 = jnp.concatenate([a1_ref[0], a1_ref[1]], axis=-1)
    a2 = a2_ref[0] + a2_ref[1]
    rc = rc_ref[...]
    lin_l = (jnp.dot(a1, wlx_ref[...], preferred_element_type=jnp.float32)
             + jnp.dot(a2, wle_ref[...], preferred_element_type=jnp.float32)) * rc
    lin_r = (jnp.dot(x_ref[...], wrx_ref[...], preferred_element_type=jnp.float32)
             + jnp.dot(ae_ref[...], wre_ref[...], preferred_element_type=jnp.float32))
    out = lin_l + lin_r + bl_ref[...]
    mu = jnp.mean(out, axis=-1, keepdims=True)
    var = jnp.mean((out - mu) ** 2, axis=-1, keepdims=True)
    out = (out - mu) * lax.rsqrt(var + 1e-5)
    out = out * g_ref[...] + b_ref[...]
    out_ref[...] = jnp.maximum(out, 0.0)


@functools.lru_cache(maxsize=None)
def _dense_kernel(N, D, F, BN):
    nb = N // BN
    H = D // 2
    full = lambda i: (0, 0)
    return pl.pallas_call(
        _dense_body,
        grid=(nb,),
        in_specs=[
            pl.BlockSpec((2, BN, H), lambda i: (0, i, 0)),
            pl.BlockSpec((2, BN, F), lambda i: (0, i, 0)),
            pl.BlockSpec((BN, F), lambda i: (i, 0)),
            pl.BlockSpec((BN, 1), lambda i: (i, 0)),
            pl.BlockSpec((BN, D), lambda i: (i, 0)),
            pl.BlockSpec((D, D), full),
            pl.BlockSpec((F, D), full),
            pl.BlockSpec((D, D), full),
            pl.BlockSpec((F, D), full),
            pl.BlockSpec((1, D), full),
            pl.BlockSpec((1, D), full),
            pl.BlockSpec((1, D), full),
        ],
        out_specs=pl.BlockSpec((BN, D), lambda i: (i, 0)),
        out_shape=jax.ShapeDtypeStruct((N, D), jnp.float32),
    )


@jax.jit
def kernel(x, edge_index, edge_attr, W_l, b_l, W_r, ln_gamma, ln_beta):
    N, D = x.shape
    E, F = edge_attr.shape
    H = D // 2

    src = edge_index[0]
    tgt = edge_index[1]
    xs = jnp.concatenate([x[:, :H], x[:, H:]], axis=0)

    zH = jnp.zeros((N, H), jnp.float32)
    zF = jnp.zeros((N, F), jnp.float32)
    z1 = jnp.zeros((N,), jnp.float32)

    aes2, cnt2 = _edge_attr_agg(N, E, F)(tgt, edge_attr, zF, z1)
    A1s = _x_gather_scatter(N, E, H)(xs, src, tgt, zH)

    ae, rc = _mean_kernel(N, F, 1000)(
        aes2.reshape(2, N, F), cnt2.reshape(2, N, 1))

    A2s = _ae_gather_scatter(N, E, F)(ae, src, tgt, zF)

    WlxT = W_l[:, :D].T
    WleT = W_l[:, D:].T
    WrxT = W_r[:, :D].T
    WreT = W_r[:, D:].T

    out = _dense_kernel(N, D, F, 500)(
        A1s.reshape(2, N, H), A2s.reshape(2, N, F), ae, rc, x,
        WlxT, WleT, WrxT, WreT,
        b_l.reshape(1, D), ln_gamma.reshape(1, D), ln_beta.reshape(1, D))
    return out


# speedup vs baseline: 4.2136x; 4.2136x over previous
import functools

import jax
import jax.numpy as jnp
from jax import lax
from jax.experimental import pallas as pl
from jax.experimental.pallas import tpu as pltpu
from jax.experimental.pallas import tpu_sc as plsc

NC = 2
NS = 16
CHUNK = 128
LW = 128


def _mesh():
    return plsc.VectorSubcoreMesh(core_axis_name="c", subcore_axis_name="s")


def _npt(N):
    return ((N + NS - 1) // NS + 15) // 16 * 16


@functools.lru_cache(maxsize=None)
def _edge_attr_agg(N, E, F):
    nchunks = E // CHUNK
    NW = NC * NS
    npt = _npt(N)
    Np = NS * npt

    @functools.partial(
        pl.kernel,
        mesh=_mesh(),
        out_type=(
            jax.ShapeDtypeStruct((NC * Np, LW), jnp.float32),
            jax.ShapeDtypeStruct((NC * Np,), jnp.float32),
        ),
        scratch_types=[
            pltpu.VMEM((CHUNK,), jnp.int32),
            pltpu.VMEM((CHUNK // 8, LW), jnp.float32),
            pltpu.VMEM((CHUNK, LW), jnp.float32),
            pltpu.VMEM((CHUNK,), jnp.float32),
            pltpu.VMEM((npt,), jnp.float32),
            pltpu.VMEM_SHARED((Np, LW), jnp.float32),
            pltpu.VMEM_SHARED((Np,), jnp.float32),
        ],
    )
    def k(tgt_hbm, attr_hbm, zW_hbm, outF_hbm, outC_hbm,
          idx_v, bu_v, rows_v, ones_v, cb_v, accF, accC):
        c = lax.axis_index("c")
        s = lax.axis_index("s")
        w = s * NC + c
        r0 = pl.multiple_of(s * npt, 8)
        pltpu.sync_copy(zW_hbm.at[pl.ds(r0, npt)], accF.at[pl.ds(r0, npt)])
        for i in range(npt // 16):
            cb_v[pl.ds(i * 16, 16)] = jnp.zeros((16,), jnp.float32)
        pltpu.sync_copy(cb_v, accC.at[pl.ds(r0, npt)])
        for i in range(CHUNK // 16):
            ones_v[pl.ds(i * 16, 16)] = jnp.full((16,), 1.0, jnp.float32)
        for j in range(CHUNK):
            for i in range(F, LW, 16):
                rows_v[j, pl.ds(i, 16)] = jnp.zeros((16,), jnp.float32)
        plsc.subcore_barrier()

        nmine = (nchunks - w + NW - 1) // NW

        def body(jj, carry):
            base = (w + jj * NW) * CHUNK
            pltpu.sync_copy(tgt_hbm.at[pl.ds(base, CHUNK)], idx_v)
            b8 = pl.multiple_of((w + jj * NW) * (CHUNK * F // LW), 8)
            pltpu.sync_copy(attr_hbm.at[pl.ds(b8, CHUNK * F // LW)], bu_v)
            for j in range(CHUNK):
                rows_v[j, pl.ds(0, F)] = bu_v[j * F // LW,
                                              pl.ds(j * F % LW, F)]
            pltpu.sync_copy(rows_v, accF.at[idx_v], add=True)
            pltpu.sync_copy(ones_v, accC.at[idx_v], add=True)
            return carry

        lax.fori_loop(0, nmine, body, 0)
        plsc.subcore_barrier()
        o0 = pl.multiple_of(c * Np + r0, 8)
        pltpu.sync_copy(accF.at[pl.ds(r0, npt)], outF_hbm.at[pl.ds(o0, npt)])
        pltpu.sync_copy(accC.at[pl.ds(r0, npt)], cb_v)
        pltpu.sync_copy(cb_v, outC_hbm.at[pl.ds(o0, npt)])

    return k


@functools.lru_cache(maxsize=None)
def _z_gather_scatter(N, E):
    nchunks = E // CHUNK
    npt = _npt(N)
    Np = NS * npt

    @functools.partial(
        pl.kernel,
        mesh=_mesh(),
        out_type=jax.ShapeDtypeStruct((NC * Np, LW), jnp.float32),
        scratch_types=[
            pltpu.VMEM((CHUNK,), jnp.int32),
            pltpu.VMEM((CHUNK,), jnp.int32),
            pltpu.VMEM((CHUNK, LW), jnp.float32),
            pltpu.SemaphoreType.DMA,
            pltpu.VMEM_SHARED((Np, LW), jnp.float32),
        ],
    )
    def k(zs_hbm, src_hbm, tgt_hbm, zW_hbm, out_hbm,
          src_v, tgt_v, rows_v, sem, acc):
        c = lax.axis_index("c")
        s = lax.axis_index("s")
        r0 = pl.multiple_of(s * npt, 8)
        pltpu.sync_copy(zW_hbm.at[pl.ds(r0, npt)], acc.at[pl.ds(r0, npt)])
        plsc.subcore_barrier()

        off = c * N
        nmine = (nchunks - s + NS - 1) // NS

        def body(jj, carry):
            base = (s + jj * NS) * CHUNK
            pltpu.sync_copy(src_hbm.at[pl.ds(base, CHUNK)], src_v)
            for i in range(CHUNK // 16):
                sl = pl.ds(i * 16, 16)
                src_v[sl] = src_v[sl] + off
            pltpu.async_copy(zs_hbm.at[src_v], rows_v, sem).wait()
            pltpu.sync_copy(tgt_hbm.at[pl.ds(base, CHUNK)], tgt_v)
            pltpu.sync_copy(rows_v, acc.at[tgt_v], add=True)
            return carry

        lax.fori_loop(0, nmine, body, 0)
        plsc.subcore_barrier()
        o0 = pl.multiple_of(c * Np + r0, 8)
        pltpu.sync_copy(acc.at[pl.ds(r0, npt)], out_hbm.at[pl.ds(o0, npt)])

    return k


def _pre_body(aesP_ref, cnt_ref, x_ref, wlx_ref, wle_ref, wrx_ref, wre_ref,
              bl_ref, zs_ref, r_ref, rc_ref, F):
    cnt = cnt_ref[0] + cnt_ref[1]
    rc = 1.0 / jnp.maximum(cnt, 1.0)
    aes = (aesP_ref[0] + aesP_ref[1])[:, :F]
    x = x_ref[...]
    z = (jnp.dot(x, wlx_ref[...], preferred_element_type=jnp.float32)
         + jnp.dot(aes, wle_ref[...], preferred_element_type=jnp.float32) * rc)
    r = (jnp.dot(x, wrx_ref[...], preferred_element_type=jnp.float32)
         + jnp.dot(aes, wre_ref[...], preferred_element_type=jnp.float32) * rc
         + bl_ref[...])
    H = z.shape[-1] // 2
    zs_ref[0] = z[:, :H]
    zs_ref[1] = z[:, H:]
    r_ref[...] = r
    rc_ref[...] = rc


@functools.lru_cache(maxsize=None)
def _pre_kernel(N, D, F, BN, Np):
    nb = N // BN
    H = D // 2
    full = lambda i: (0, 0)
    return pl.pallas_call(
        functools.partial(_pre_body, F=F),
        grid=(nb,),
        in_specs=[
            pl.BlockSpec((2, BN, LW), lambda i: (0, i, 0)),
            pl.BlockSpec((2, BN, 1), lambda i: (0, i, 0)),
            pl.BlockSpec((BN, D), lambda i: (i, 0)),
            pl.BlockSpec((D, D), full),
            pl.BlockSpec((F, D), full),
            pl.BlockSpec((D, D), full),
            pl.BlockSpec((F, D), full),
            pl.BlockSpec((1, D), full),
        ],
        out_specs=[
            pl.BlockSpec((2, BN, H), lambda i: (0, i, 0)),
            pl.BlockSpec((BN, D), lambda i: (i, 0)),
            pl.BlockSpec((BN, 1), lambda i: (i, 0)),
        ],
        out_shape=[
            jax.ShapeDtypeStruct((2, N, D // 2), jnp.float32),
            jax.ShapeDtypeStruct((N, D), jnp.float32),
            jax.ShapeDtypeStruct((N, 1), jnp.float32),
        ],
    )


def _post_body(a_ref, rc_ref, r_ref, g_ref, b_ref, out_ref):
    a = jnp.concatenate([a_ref[0], a_ref[1]], axis=-1)
    out = a * rc_ref[...] + r_ref[...]
    mu = jnp.mean(out, axis=-1, keepdims=True)
    var = jnp.mean((out - mu) ** 2, axis=-1, keepdims=True)
    out = (out - mu) * lax.rsqrt(var + 1e-5)
    out = out * g_ref[...] + b_ref[...]
    out_ref[...] = jnp.maximum(out, 0.0)


@functools.lru_cache(maxsize=None)
def _post_kernel(N, D, BN, Np):
    nb = N // BN
    full = lambda i: (0, 0)
    return pl.pallas_call(
        _post_body,
        grid=(nb,),
        in_specs=[
            pl.BlockSpec((2, BN, D // 2), lambda i: (0, i, 0)),
            pl.BlockSpec((BN, 1), lambda i: (i, 0)),
            pl.BlockSpec((BN, D), lambda i: (i, 0)),
            pl.BlockSpec((1, D), full),
            pl.BlockSpec((1, D), full),
        ],
        out_specs=pl.BlockSpec((BN, D), lambda i: (i, 0)),
        out_shape=jax.ShapeDtypeStruct((N, D), jnp.float32),
    )


@jax.jit
def kernel(x, edge_index, edge_attr, W_l, b_l, W_r, ln_gamma, ln_beta):
    N, D = x.shape
    E, F = edge_attr.shape
    H = D // 2
    Np = NS * _npt(N)

    src = edge_index[0]
    tgt = edge_index[1]
    attr128 = edge_attr.reshape(E * F // LW, LW)
    zW = jnp.zeros((Np, LW), jnp.float32)

    aesP, cnt2 = _edge_attr_agg(N, E, F)(tgt, attr128, zW)

    WlxT = W_l[:, :D].T
    WleT = W_l[:, D:].T
    WrxT = W_r[:, :D].T
    WreT = W_r[:, D:].T

    zs, r, rc = _pre_kernel(N, D, F, 1000, Np)(
        aesP.reshape(2, Np, LW), cnt2.reshape(2, Np, 1), x,
        WlxT, WleT, WrxT, WreT, b_l.reshape(1, D))

    A = _z_gather_scatter(N, E)(zs.reshape(2 * N, H), src, tgt, zW)

    out = _post_kernel(N, D, 1000, Np)(
        A.reshape(2, Np, H), rc, r,
        ln_gamma.reshape(1, D), ln_beta.reshape(1, D))
    return out


# speedup vs baseline: 5.9616x; 1.4149x over previous
import functools

import jax
import jax.numpy as jnp
from jax import lax
from jax.experimental import pallas as pl
from jax.experimental.pallas import tpu as pltpu
from jax.experimental.pallas import tpu_sc as plsc

NC = 2
NS = 16
CHUNK = 128
LW = 128


def _mesh():
    return plsc.VectorSubcoreMesh(core_axis_name="c", subcore_axis_name="s")


def _npt(N):
    return ((N + NS - 1) // NS + 15) // 16 * 16


@functools.lru_cache(maxsize=None)
def _edge_attr_agg(N, E, F):
    nchunks = E // CHUNK
    NW = NC * NS
    npt = _npt(N)
    Np = NS * npt

    @functools.partial(
        pl.kernel,
        mesh=_mesh(),
        out_type=(
            jax.ShapeDtypeStruct((NC * Np, LW), jnp.float32),
            jax.ShapeDtypeStruct((NC * Np,), jnp.float32),
        ),
        scratch_types=[
            pltpu.VMEM((CHUNK,), jnp.int32),
            pltpu.VMEM((CHUNK // 8, LW), jnp.float32),
            pltpu.VMEM((CHUNK, LW), jnp.float32),
            pltpu.VMEM((CHUNK,), jnp.float32),
            pltpu.VMEM((npt,), jnp.float32),
            pltpu.VMEM_SHARED((Np, LW), jnp.float32),
            pltpu.VMEM_SHARED((Np,), jnp.float32),
        ],
    )
    def k(tgt_hbm, attr_hbm, zW_hbm, outF_hbm, outC_hbm,
          idx_v, bu_v, rows_v, ones_v, cb_v, accF, accC):
        c = lax.axis_index("c")
        s = lax.axis_index("s")
        w = s * NC + c
        r0 = pl.multiple_of(s * npt, 8)
        pltpu.sync_copy(zW_hbm.at[pl.ds(r0, npt)], accF.at[pl.ds(r0, npt)])
        for i in range(npt // 16):
            cb_v[pl.ds(i * 16, 16)] = jnp.zeros((16,), jnp.float32)
        pltpu.sync_copy(cb_v, accC.at[pl.ds(r0, npt)])
        for i in range(CHUNK // 16):
            ones_v[pl.ds(i * 16, 16)] = jnp.full((16,), 1.0, jnp.float32)
        for j in range(CHUNK):
            for i in range(F, LW, 16):
                rows_v[j, pl.ds(i, 16)] = jnp.zeros((16,), jnp.float32)
        plsc.subcore_barrier()

        nmine = (nchunks - w + NW - 1) // NW

        def body(jj, carry):
            base = (w + jj * NW) * CHUNK
            pltpu.sync_copy(tgt_hbm.at[pl.ds(base, CHUNK)], idx_v)
            b8 = pl.multiple_of((w + jj * NW) * (CHUNK * F // LW), 8)
            pltpu.sync_copy(attr_hbm.at[pl.ds(b8, CHUNK * F // LW)], bu_v)
            for j in range(CHUNK):
                rows_v[j, pl.ds(0, F)] = bu_v[j * F // LW,
                                              pl.ds(j * F % LW, F)]
            pltpu.sync_copy(rows_v, accF.at[idx_v], add=True)
            pltpu.sync_copy(ones_v, accC.at[idx_v], add=True)
            return carry

        lax.fori_loop(0, nmine, body, 0)
        plsc.subcore_barrier()
        o0 = pl.multiple_of(c * Np + r0, 8)
        pltpu.sync_copy(accF.at[pl.ds(r0, npt)], outF_hbm.at[pl.ds(o0, npt)])
        pltpu.sync_copy(accC.at[pl.ds(r0, npt)], cb_v)
        pltpu.sync_copy(cb_v, outC_hbm.at[pl.ds(o0, npt)])

    return k


@functools.lru_cache(maxsize=None)
def _z_gather_scatter(N, E):
    npt = _npt(N)
    Np = NS * npt
    ept = E // NS
    nfull = ept // CHUNK
    tail = ept - nfull * CHUNK
    assert nfull % 2 == 0 and ept * NS == E and tail % 8 == 0

    @functools.partial(
        pl.kernel,
        mesh=_mesh(),
        out_type=jax.ShapeDtypeStruct((NC * Np, LW), jnp.float32),
        scratch_types=[
            pltpu.VMEM((CHUNK,), jnp.int32),
            pltpu.VMEM((CHUNK,), jnp.int32),
            pltpu.VMEM((CHUNK,), jnp.int32),
            pltpu.VMEM((CHUNK,), jnp.int32),
            pltpu.VMEM((CHUNK,), jnp.int32),
            pltpu.VMEM((CHUNK,), jnp.int32),
            pltpu.VMEM((CHUNK,), jnp.int32),
            pltpu.VMEM((CHUNK,), jnp.int32),
            pltpu.VMEM((CHUNK, LW), jnp.float32),
            pltpu.VMEM((CHUNK, LW), jnp.float32),
            pltpu.SemaphoreType.DMA,
            pltpu.SemaphoreType.DMA,
            pltpu.SemaphoreType.DMA,
            pltpu.SemaphoreType.DMA,
            pltpu.VMEM_SHARED((Np, LW), jnp.float32),
        ],
    )
    def k(zs_hbm, src_hbm, tgt_hbm, zW_hbm, out_hbm,
          si0, si1, ti0, ti1, gi0, gi1, st0, st1, rows0, rows1,
          smi0, smi1, smg0, smg1, acc):
        c = lax.axis_index("c")
        s = lax.axis_index("s")
        r0 = pl.multiple_of(s * npt, 8)
        pltpu.sync_copy(zW_hbm.at[pl.ds(r0, npt)], acc.at[pl.ds(r0, npt)])
        plsc.subcore_barrier()

        e0 = pl.multiple_of(s * ept, 8)
        off = c * N
        si = (si0, si1)
        ti = (ti0, ti1)
        gi = (gi0, gi1)
        st = (st0, st1)
        rows = (rows0, rows1)
        smi = (smi0, smi1)
        smg = (smg0, smg1)

        def start_idx(j, b, n=CHUNK):
            base = e0 + j * CHUNK
            pltpu.async_copy(src_hbm.at[pl.ds(base, n)],
                             si[b].at[pl.ds(0, n)], smi[b])
            pltpu.async_copy(tgt_hbm.at[pl.ds(base, n)],
                             ti[b].at[pl.ds(0, n)], smi[b])

        def wait_idx(j, b, n=CHUNK):
            base = e0 + j * CHUNK
            pltpu.make_async_copy(src_hbm.at[pl.ds(base, n)],
                                  si[b].at[pl.ds(0, n)], smi[b]).wait()
            pltpu.make_async_copy(tgt_hbm.at[pl.ds(base, n)],
                                  ti[b].at[pl.ds(0, n)], smi[b]).wait()

        def prep(b, n=CHUNK):
            for i in range(n // 16):
                sl = pl.ds(i * 16, 16)
                gi[b][sl] = si[b][sl] + off
                st[b][sl] = ti[b][sl]

        def start_g(b, n=CHUNK):
            pltpu.async_copy(zs_hbm.at[gi[b].at[pl.ds(0, n)]],
                             rows[b].at[pl.ds(0, n)], smg[b])

        def wait_g(b, n=CHUNK):
            pltpu.make_async_copy(zs_hbm.at[gi[b].at[pl.ds(0, n)]],
                                  rows[b].at[pl.ds(0, n)], smg[b]).wait()

        def scatter(b, n=CHUNK):
            pltpu.sync_copy(rows[b].at[pl.ds(0, n)],
                            acc.at[st[b].at[pl.ds(0, n)]], add=True)

        start_idx(0, 0)
        start_idx(1, 1)
        wait_idx(0, 0)
        prep(0)
        start_idx(2, 0)
        start_g(0)

        def body(jj, carry):
            j1 = jj * 2 + 1
            wait_idx(j1, 1)
            prep(1)

            @pl.when(j1 + 2 < nfull)
            def _():
                start_idx(j1 + 2, 1)

            start_g(1)
            wait_g(0)
            scatter(0)

            j2 = j1 + 1
            wait_idx(j2, 0)
            prep(0)

            @pl.when(j2 + 2 < nfull)
            def _():
                start_idx(j2 + 2, 0)

            @pl.when(j2 < nfull)
            def _():
                start_g(0)

            wait_g(1)
            scatter(1)
            return carry

        lax.fori_loop(0, (nfull - 2) // 2, body, 0)

        jl = nfull - 1
        wait_idx(jl, 1)
        prep(1)
        if tail:
            start_idx(nfull, 1, tail)
        start_g(1)
        wait_g(0)
        scatter(0)
        wait_g(1)
        scatter(1)

        if tail:
            wait_idx(nfull, 1, tail)
            prep(1, tail)
            start_g(1, tail)
            wait_g(1, tail)
            scatter(1, tail)

        plsc.subcore_barrier()
        o0 = pl.multiple_of(c * Np + r0, 8)
        pltpu.sync_copy(acc.at[pl.ds(r0, npt)], out_hbm.at[pl.ds(o0, npt)])

    return k


def _pre_body(aesP_ref, cnt_ref, x_ref, wlx_ref, wle_ref, wrx_ref, wre_ref,
              bl_ref, zs_ref, r_ref, rc_ref, F):
    cnt = cnt_ref[0] + cnt_ref[1]
    rc = 1.0 / jnp.maximum(cnt, 1.0)
    aes = (aesP_ref[0] + aesP_ref[1])[:, :F]
    x = x_ref[...]
    z = (jnp.dot(x, wlx_ref[...], preferred_element_type=jnp.float32)
         + jnp.dot(aes, wle_ref[...], preferred_element_type=jnp.float32) * rc)
    r = (jnp.dot(x, wrx_ref[...], preferred_element_type=jnp.float32)
         + jnp.dot(aes, wre_ref[...], preferred_element_type=jnp.float32) * rc
         + bl_ref[...])
    H = z.shape[-1] // 2
    zs_ref[0] = z[:, :H]
    zs_ref[1] = z[:, H:]
    r_ref[...] = r
    rc_ref[...] = rc


@functools.lru_cache(maxsize=None)
def _pre_kernel(N, D, F, BN, Np):
    nb = N // BN
    H = D // 2
    full = lambda i: (0, 0)
    return pl.pallas_call(
        functools.partial(_pre_body, F=F),
        grid=(nb,),
        in_specs=[
            pl.BlockSpec((2, BN, LW), lambda i: (0, i, 0)),
            pl.BlockSpec((2, BN, 1), lambda i: (0, i, 0)),
            pl.BlockSpec((BN, D), lambda i: (i, 0)),
            pl.BlockSpec((D, D), full),
            pl.BlockSpec((F, D), full),
            pl.BlockSpec((D, D), full),
            pl.BlockSpec((F, D), full),
            pl.BlockSpec((1, D), full),
        ],
        out_specs=[
            pl.BlockSpec((2, BN, H), lambda i: (0, i, 0)),
            pl.BlockSpec((BN, D), lambda i: (i, 0)),
            pl.BlockSpec((BN, 1), lambda i: (i, 0)),
        ],
        out_shape=[
            jax.ShapeDtypeStruct((2, N, D // 2), jnp.float32),
            jax.ShapeDtypeStruct((N, D), jnp.float32),
            jax.ShapeDtypeStruct((N, 1), jnp.float32),
        ],
    )


def _post_body(a_ref, rc_ref, r_ref, g_ref, b_ref, out_ref):
    a = jnp.concatenate([a_ref[0], a_ref[1]], axis=-1)
    out = a * rc_ref[...] + r_ref[...]
    mu = jnp.mean(out, axis=-1, keepdims=True)
    var = jnp.mean((out - mu) ** 2, axis=-1, keepdims=True)
    out = (out - mu) * lax.rsqrt(var + 1e-5)
    out = out * g_ref[...] + b_ref[...]
    out_ref[...] = jnp.maximum(out, 0.0)


@functools.lru_cache(maxsize=None)
def _post_kernel(N, D, BN, Np):
    nb = N // BN
    full = lambda i: (0, 0)
    return pl.pallas_call(
        _post_body,
        grid=(nb,),
        in_specs=[
            pl.BlockSpec((2, BN, D // 2), lambda i: (0, i, 0)),
            pl.BlockSpec((BN, 1), lambda i: (i, 0)),
            pl.BlockSpec((BN, D), lambda i: (i, 0)),
            pl.BlockSpec((1, D), full),
            pl.BlockSpec((1, D), full),
        ],
        out_specs=pl.BlockSpec((BN, D), lambda i: (i, 0)),
        out_shape=jax.ShapeDtypeStruct((N, D), jnp.float32),
    )


@jax.jit
def kernel(x, edge_index, edge_attr, W_l, b_l, W_r, ln_gamma, ln_beta):
    N, D = x.shape
    E, F = edge_attr.shape
    H = D // 2
    Np = NS * _npt(N)

    src = edge_index[0]
    tgt = edge_index[1]
    attr128 = edge_attr.reshape(E * F // LW, LW)
    zW = jnp.zeros((Np, LW), jnp.float32)

    aesP, cnt2 = _edge_attr_agg(N, E, F)(tgt, attr128, zW)

    WlxT = W_l[:, :D].T
    WleT = W_l[:, D:].T
    WrxT = W_r[:, :D].T
    WreT = W_r[:, D:].T

    zs, r, rc = _pre_kernel(N, D, F, 1000, Np)(
        aesP.reshape(2, Np, LW), cnt2.reshape(2, Np, 1), x,
        WlxT, WleT, WrxT, WreT, b_l.reshape(1, D))

    A = _z_gather_scatter(N, E)(zs.reshape(2 * N, H), src, tgt, zW)

    out = _post_kernel(N, D, 1000, Np)(
        A.reshape(2, Np, H), rc, r,
        ln_gamma.reshape(1, D), ln_beta.reshape(1, D))
    return out


# speedup vs baseline: 7.1779x; 1.2040x over previous
import functools

import jax
import jax.numpy as jnp
from jax import lax
from jax.experimental import pallas as pl
from jax.experimental.pallas import tpu as pltpu
from jax.experimental.pallas import tpu_sc as plsc

NC = 2
NS = 16
CHUNK = 128
LW = 128


def _mesh():
    return plsc.VectorSubcoreMesh(core_axis_name="c", subcore_axis_name="s")


def _npt(N):
    return ((N + NS - 1) // NS + 15) // 16 * 16


@functools.lru_cache(maxsize=None)
def _edge_attr_agg(N, E, F):
    NW = NC * NS
    npt = _npt(N)
    Np = NS * npt
    nchunks = E // CHUNK
    nfull = nchunks // NW
    nextra = nchunks - nfull * NW
    ar = CHUNK * F // LW
    assert nfull >= 4 and nfull % 2 == 1 or nfull % 2 == 0

    @functools.partial(
        pl.kernel,
        mesh=_mesh(),
        out_type=jax.ShapeDtypeStruct((NC * Np, LW), jnp.float32),
        scratch_types=[
            pltpu.VMEM((CHUNK,), jnp.int32),
            pltpu.VMEM((CHUNK,), jnp.int32),
            pltpu.VMEM((ar, LW), jnp.float32),
            pltpu.VMEM((ar, LW), jnp.float32),
            pltpu.VMEM((CHUNK,), jnp.int32),
            pltpu.VMEM((CHUNK,), jnp.int32),
            pltpu.VMEM((CHUNK, LW), jnp.float32),
            pltpu.VMEM((CHUNK, LW), jnp.float32),
            pltpu.SemaphoreType.DMA,
            pltpu.SemaphoreType.DMA,
            pltpu.VMEM_SHARED((Np, LW), jnp.float32),
        ],
    )
    def k(tgt_hbm, attr_hbm, zW_hbm, outF_hbm,
          ti0, ti1, bu0, bu1, st0, st1, rows0, rows1, smi0, smi1, accF):
        c = lax.axis_index("c")
        s = lax.axis_index("s")
        w = s * NC + c
        r0 = pl.multiple_of(s * npt, 8)
        pltpu.sync_copy(zW_hbm.at[pl.ds(r0, npt)], accF.at[pl.ds(r0, npt)])

        ti = (ti0, ti1)
        bu = (bu0, bu1)
        st = (st0, st1)
        rows = (rows0, rows1)
        smi = (smi0, smi1)

        cones = jnp.where(lax.iota(jnp.int32, 16) == 0,
                          jnp.float32(1.0), jnp.float32(0.0))
        for r in rows:
            for j in range(CHUNK):
                r[j, pl.ds(F, 16)] = cones
                for i in range(F + 16, LW, 16):
                    r[j, pl.ds(i, 16)] = jnp.zeros((16,), jnp.float32)
        plsc.subcore_barrier()

        def chunk_id(j):
            return w + j * NW

        def start_idx(j, b):
            q = chunk_id(j)
            base = pl.multiple_of(q * CHUNK, 8)
            ab = pl.multiple_of(q * ar, 8)
            pltpu.async_copy(tgt_hbm.at[pl.ds(base, CHUNK)], ti[b], smi[b])
            pltpu.async_copy(attr_hbm.at[pl.ds(ab, ar)], bu[b], smi[b])

        def wait_idx(j, b):
            q = chunk_id(j)
            base = pl.multiple_of(q * CHUNK, 8)
            ab = pl.multiple_of(q * ar, 8)
            pltpu.make_async_copy(tgt_hbm.at[pl.ds(base, CHUNK)], ti[b],
                                  smi[b]).wait()
            pltpu.make_async_copy(attr_hbm.at[pl.ds(ab, ar)], bu[b],
                                  smi[b]).wait()

        def prep(b):
            for i in range(CHUNK // 16):
                sl = pl.ds(i * 16, 16)
                st[b][sl] = ti[b][sl]
            for j in range(CHUNK):
                rows[b][j, pl.ds(0, F)] = bu[b][j * F // LW,
                                                pl.ds(j * F % LW, F)]

        def scatter(b):
            pltpu.sync_copy(rows[b], accF.at[st[b]], add=True)

        start_idx(0, 0)
        start_idx(1, 1)
        wait_idx(0, 0)
        prep(0)
        start_idx(2, 0)

        def body(jj, carry):
            j1 = jj * 2 + 1
            wait_idx(j1, 1)
            prep(1)

            @pl.when(j1 + 2 < nfull)
            def _():
                start_idx(j1 + 2, 1)

            scatter(0)
            j2 = j1 + 1
            wait_idx(j2, 0)
            prep(0)

            @pl.when(j2 + 2 < nfull)
            def _():
                start_idx(j2 + 2, 0)

            scatter(1)
            return carry

        npairs = (nfull - 1) // 2
        lax.fori_loop(0, npairs, body, 0)

        if nfull % 2 == 0:
            jl = nfull - 1
            wait_idx(jl, 1)
            prep(1)
            scatter(0)
            scatter(1)
        else:
            scatter(0)

        @pl.when(w < nextra)
        def _():
            q = nfull * NW + w
            base = pl.multiple_of(q * CHUNK, 8)
            ab = pl.multiple_of(q * ar, 8)
            pltpu.sync_copy(tgt_hbm.at[pl.ds(base, CHUNK)], ti0)
            pltpu.sync_copy(attr_hbm.at[pl.ds(ab, ar)], bu0)
            prep(0)
            scatter(0)

        plsc.subcore_barrier()
        o0 = pl.multiple_of(c * Np + r0, 8)
        pltpu.sync_copy(accF.at[pl.ds(r0, npt)], outF_hbm.at[pl.ds(o0, npt)])

    return k


@functools.lru_cache(maxsize=None)
def _z_gather_scatter(N, E):
    npt = _npt(N)
    Np = NS * npt
    ept = E // NS
    nfull = ept // CHUNK
    tail = ept - nfull * CHUNK
    assert nfull % 2 == 0 and ept * NS == E and tail % 8 == 0

    @functools.partial(
        pl.kernel,
        mesh=_mesh(),
        out_type=jax.ShapeDtypeStruct((NC * Np, LW), jnp.float32),
        scratch_types=[
            pltpu.VMEM((CHUNK,), jnp.int32),
            pltpu.VMEM((CHUNK,), jnp.int32),
            pltpu.VMEM((CHUNK,), jnp.int32),
            pltpu.VMEM((CHUNK,), jnp.int32),
            pltpu.VMEM((CHUNK,), jnp.int32),
            pltpu.VMEM((CHUNK,), jnp.int32),
            pltpu.VMEM((CHUNK,), jnp.int32),
            pltpu.VMEM((CHUNK,), jnp.int32),
            pltpu.VMEM((CHUNK, LW), jnp.float32),
            pltpu.VMEM((CHUNK, LW), jnp.float32),
            pltpu.SemaphoreType.DMA,
            pltpu.SemaphoreType.DMA,
            pltpu.SemaphoreType.DMA,
            pltpu.SemaphoreType.DMA,
            pltpu.VMEM_SHARED((Np, LW), jnp.float32),
        ],
    )
    def k(zs_hbm, src_hbm, tgt_hbm, zW_hbm, out_hbm,
          si0, si1, ti0, ti1, gi0, gi1, st0, st1, rows0, rows1,
          smi0, smi1, smg0, smg1, acc):
        c = lax.axis_index("c")
        s = lax.axis_index("s")
        r0 = pl.multiple_of(s * npt, 8)
        pltpu.sync_copy(zW_hbm.at[pl.ds(r0, npt)], acc.at[pl.ds(r0, npt)])
        plsc.subcore_barrier()

        e0 = pl.multiple_of(s * ept, 8)
        off = c * N
        si = (si0, si1)
        ti = (ti0, ti1)
        gi = (gi0, gi1)
        st = (st0, st1)
        rows = (rows0, rows1)
        smi = (smi0, smi1)
        smg = (smg0, smg1)

        def start_idx(j, b, n=CHUNK):
            base = e0 + j * CHUNK
            pltpu.async_copy(src_hbm.at[pl.ds(base, n)],
                             si[b].at[pl.ds(0, n)], smi[b])
            pltpu.async_copy(tgt_hbm.at[pl.ds(base, n)],
                             ti[b].at[pl.ds(0, n)], smi[b])

        def wait_idx(j, b, n=CHUNK):
            base = e0 + j * CHUNK
            pltpu.make_async_copy(src_hbm.at[pl.ds(base, n)],
                                  si[b].at[pl.ds(0, n)], smi[b]).wait()
            pltpu.make_async_copy(tgt_hbm.at[pl.ds(base, n)],
                                  ti[b].at[pl.ds(0, n)], smi[b]).wait()

        def prep(b, n=CHUNK):
            for i in range(n // 16):
                sl = pl.ds(i * 16, 16)
                gi[b][sl] = si[b][sl] + off
                st[b][sl] = ti[b][sl]

        def start_g(b, n=CHUNK):
            pltpu.async_copy(zs_hbm.at[gi[b].at[pl.ds(0, n)]],
                             rows[b].at[pl.ds(0, n)], smg[b])

        def wait_g(b, n=CHUNK):
            pltpu.make_async_copy(zs_hbm.at[gi[b].at[pl.ds(0, n)]],
                                  rows[b].at[pl.ds(0, n)], smg[b]).wait()

        def scatter(b, n=CHUNK):
            pltpu.sync_copy(rows[b].at[pl.ds(0, n)],
                            acc.at[st[b].at[pl.ds(0, n)]], add=True)

        start_idx(0, 0)
        start_idx(1, 1)
        wait_idx(0, 0)
        prep(0)
        start_idx(2, 0)
        start_g(0)

        def body(jj, carry):
            j1 = jj * 2 + 1
            wait_idx(j1, 1)
            prep(1)

            @pl.when(j1 + 2 < nfull)
            def _():
                start_idx(j1 + 2, 1)

            start_g(1)
            wait_g(0)
            scatter(0)

            j2 = j1 + 1
            wait_idx(j2, 0)
            prep(0)

            @pl.when(j2 + 2 < nfull)
            def _():
                start_idx(j2 + 2, 0)

            @pl.when(j2 < nfull)
            def _():
                start_g(0)

            wait_g(1)
            scatter(1)
            return carry

        lax.fori_loop(0, (nfull - 2) // 2, body, 0)

        jl = nfull - 1
        wait_idx(jl, 1)
        prep(1)
        if tail:
            start_idx(nfull, 1, tail)
        start_g(1)
        wait_g(0)
        scatter(0)
        wait_g(1)
        scatter(1)

        if tail:
            wait_idx(nfull, 1, tail)
            prep(1, tail)
            start_g(1, tail)
            wait_g(1, tail)
            scatter(1, tail)

        plsc.subcore_barrier()
        o0 = pl.multiple_of(c * Np + r0, 8)
        pltpu.sync_copy(acc.at[pl.ds(r0, npt)], out_hbm.at[pl.ds(o0, npt)])

    return k


def _pre_body(aesP_ref, x_ref, wlx_ref, wle_ref, wrx_ref, wre_ref,
              bl_ref, zs_ref, r_ref, rc_ref, F):
    aesc = aesP_ref[0] + aesP_ref[1]
    cnt = aesc[:, F:F + 1]
    rc = 1.0 / jnp.maximum(cnt, 1.0)
    aes = aesc[:, :F]
    x = x_ref[...]
    z = (jnp.dot(x, wlx_ref[...], preferred_element_type=jnp.float32)
         + jnp.dot(aes, wle_ref[...], preferred_element_type=jnp.float32) * rc)
    r = (jnp.dot(x, wrx_ref[...], preferred_element_type=jnp.float32)
         + jnp.dot(aes, wre_ref[...], preferred_element_type=jnp.float32) * rc
         + bl_ref[...])
    H = z.shape[-1] // 2
    zs_ref[0] = z[:, :H]
    zs_ref[1] = z[:, H:]
    r_ref[...] = r
    rc_ref[...] = rc


@functools.lru_cache(maxsize=None)
def _pre_kernel(N, D, F, BN, Np):
    nb = N // BN
    H = D // 2
    full = lambda i: (0, 0)
    return pl.pallas_call(
        functools.partial(_pre_body, F=F),
        grid=(nb,),
        in_specs=[
            pl.BlockSpec((2, BN, LW), lambda i: (0, i, 0)),
            pl.BlockSpec((BN, D), lambda i: (i, 0)),
            pl.BlockSpec((D, D), full),
            pl.BlockSpec((F, D), full),
            pl.BlockSpec((D, D), full),
            pl.BlockSpec((F, D), full),
            pl.BlockSpec((1, D), full),
        ],
        out_specs=[
            pl.BlockSpec((2, BN, H), lambda i: (0, i, 0)),
            pl.BlockSpec((BN, D), lambda i: (i, 0)),
            pl.BlockSpec((BN, 1), lambda i: (i, 0)),
        ],
        out_shape=[
            jax.ShapeDtypeStruct((2, N, D // 2), jnp.float32),
            jax.ShapeDtypeStruct((N, D), jnp.float32),
            jax.ShapeDtypeStruct((N, 1), jnp.float32),
        ],
    )


def _post_body(a_ref, rc_ref, r_ref, g_ref, b_ref, out_ref):
    a = jnp.concatenate([a_ref[0], a_ref[1]], axis=-1)
    out = a * rc_ref[...] + r_ref[...]
    mu = jnp.mean(out, axis=-1, keepdims=True)
    var = jnp.mean((out - mu) ** 2, axis=-1, keepdims=True)
    out = (out - mu) * lax.rsqrt(var + 1e-5)
    out = out * g_ref[...] + b_ref[...]
    out_ref[...] = jnp.maximum(out, 0.0)


@functools.lru_cache(maxsize=None)
def _post_kernel(N, D, BN, Np):
    nb = N // BN
    full = lambda i: (0, 0)
    return pl.pallas_call(
        _post_body,
        grid=(nb,),
        in_specs=[
            pl.BlockSpec((2, BN, D // 2), lambda i: (0, i, 0)),
            pl.BlockSpec((BN, 1), lambda i: (i, 0)),
            pl.BlockSpec((BN, D), lambda i: (i, 0)),
            pl.BlockSpec((1, D), full),
            pl.BlockSpec((1, D), full),
        ],
        out_specs=pl.BlockSpec((BN, D), lambda i: (i, 0)),
        out_shape=jax.ShapeDtypeStruct((N, D), jnp.float32),
    )


@jax.jit
def kernel(x, edge_index, edge_attr, W_l, b_l, W_r, ln_gamma, ln_beta):
    N, D = x.shape
    E, F = edge_attr.shape
    H = D // 2
    Np = NS * _npt(N)

    src = edge_index[0]
    tgt = edge_index[1]
    attr128 = edge_attr.reshape(E * F // LW, LW)
    zW = jnp.zeros((Np, LW), jnp.float32)

    aesP = _edge_attr_agg(N, E, F)(tgt, attr128, zW)

    WlxT = W_l[:, :D].T
    WleT = W_l[:, D:].T
    WrxT = W_r[:, :D].T
    WreT = W_r[:, D:].T

    zs, r, rc = _pre_kernel(N, D, F, 1000, Np)(
        aesP.reshape(2, Np, LW), x,
        WlxT, WleT, WrxT, WreT, b_l.reshape(1, D))

    A = _z_gather_scatter(N, E)(zs.reshape(2 * N, H), src, tgt, zW)

    out = _post_kernel(N, D, 1000, Np)(
        A.reshape(2, Np, H), rc, r,
        ln_gamma.reshape(1, D), ln_beta.reshape(1, D))
    return out
